# baseline (speedup 1.0000x reference)
import functools

import jax
import jax.numpy as jnp
from jax import lax
from jax.experimental import pallas as pl
from jax.experimental.pallas import tpu as pltpu
from jax.experimental.pallas import tpu_sc as plsc

_N = 10000
_E = 320000
_D = 128
_R = 7
_H = 128
_B = 16
_ENC = 2 * _H

_NSUB = 16
_CH = 128
_GRP = 16
_NGRP = 10
_NCH = _GRP * _NGRP
_EPAD = _NSUB * _NCH * _CH
_ZROWS = 640
_NACC = _NSUB * _ZROWS
_ROWS_PER_TILE = _N // _NSUB

_RB = 1000


def _matmul_body(x_ref, w_ref, o_ref):
    o_ref[...] = jnp.dot(x_ref[...], w_ref[...], preferred_element_type=jnp.float32)


def _row_matmul(x, w):
    m, k = x.shape
    n = w.shape[1]
    return pl.pallas_call(
        _matmul_body,
        grid=(m // _RB,),
        in_specs=[
            pl.BlockSpec((_RB, k), lambda b: (b, 0)),
            pl.BlockSpec((k, n), lambda b: (0, 0)),
        ],
        out_specs=pl.BlockSpec((_RB, n), lambda b: (b, 0)),
        out_shape=jax.ShapeDtypeStruct((m, n), jnp.float32),
    )(x, w)


def _combine_body(x_ref, agg_ref, ws_ref, b_ref, wf_ref, h_ref, t_ref):
    h = jnp.maximum(
        jnp.dot(x_ref[...], ws_ref[...], preferred_element_type=jnp.float32)
        + b_ref[...] + agg_ref[...], 0.0)
    h_ref[...] = h
    t_ref[...] = jnp.dot(h, wf_ref[...], preferred_element_type=jnp.float32)


def _combine_stage(x, agg, w_self, b_row, w_flat):
    m, k = x.shape
    n = w_flat.shape[1]
    return pl.pallas_call(
        _combine_body,
        grid=(m // _RB,),
        in_specs=[
            pl.BlockSpec((_RB, k), lambda b: (b, 0)),
            pl.BlockSpec((_RB, _H), lambda b: (b, 0)),
            pl.BlockSpec((k, _H), lambda b: (0, 0)),
            pl.BlockSpec((1, _H), lambda b: (0, 0)),
            pl.BlockSpec((_H, n), lambda b: (0, 0)),
        ],
        out_specs=[
            pl.BlockSpec((_RB, _H), lambda b: (b, 0)),
            pl.BlockSpec((_RB, n), lambda b: (b, 0)),
        ],
        out_shape=[
            jax.ShapeDtypeStruct((m, _H), jnp.float32),
            jax.ShapeDtypeStruct((m, n), jnp.float32),
        ],
    )(x, agg, w_self, b_row, w_flat)


def _final_body(h1_ref, agg_ref, ws_ref, b_ref, bf_ref, wh1a_ref, wh1b_ref,
                bh1_ref, wh2_ref, bh2_ref, o_ref, gacc):
    b = pl.program_id(0)
    nb = pl.num_programs(0)
    g = b // (nb // 2)

    @pl.when(b == 0)
    def _():
        gacc[0] = jnp.zeros_like(gacc[0])
        gacc[1] = jnp.zeros_like(gacc[1])

    h1 = h1_ref[...]
    h2 = jnp.maximum(
        jnp.dot(h1, ws_ref[...], preferred_element_type=jnp.float32)
        + b_ref[...] + agg_ref[...], 0.0)
    feats = jnp.concatenate([h1, h2], axis=1)
    lane = lax.broadcasted_iota(jnp.int32, (_RB, 128), 1).astype(jnp.float32)
    oh = (bf_ref[...] == lane).astype(jnp.float32)
    contrib = lax.dot_general(oh, feats, (((0,), (0,)), ((), ())),
                              preferred_element_type=jnp.float32)
    gacc[g] = gacc[g] + contrib

    @pl.when(b == nb - 1)
    def _():
        hid = jnp.maximum(
            jnp.dot(gacc[0], wh1a_ref[...], preferred_element_type=jnp.float32)
            + jnp.dot(gacc[1], wh1b_ref[...], preferred_element_type=jnp.float32)
            + bh1_ref[...], 0.0)
        o_ref[...] = jnp.dot(hid, wh2_ref[...],
                             preferred_element_type=jnp.float32) + bh2_ref[...]


def _final_stage(h1s, agg2s, w_self2, b2_row, batchf, wh1a, wh1b, bh1_row,
                 wh2p, bh2b):
    m = h1s.shape[0]
    return pl.pallas_call(
        _final_body,
        grid=(m // _RB,),
        in_specs=[
            pl.BlockSpec((_RB, _H), lambda b: (b, 0)),
            pl.BlockSpec((_RB, _H), lambda b: (b, 0)),
            pl.BlockSpec((_H, _H), lambda b: (0, 0)),
            pl.BlockSpec((1, _H), lambda b: (0, 0)),
            pl.BlockSpec((_RB, 128), lambda b: (b, 0)),
            pl.BlockSpec((_ENC, 128), lambda b: (0, 0)),
            pl.BlockSpec((_ENC, 128), lambda b: (0, 0)),
            pl.BlockSpec((1, 128), lambda b: (0, 0)),
            pl.BlockSpec((128, 128), lambda b: (0, 0)),
            pl.BlockSpec((1, 128), lambda b: (0, 0)),
        ],
        out_specs=pl.BlockSpec((128, 128), lambda b: (0, 0)),
        out_shape=jax.ShapeDtypeStruct((128, 128), jnp.float32),
        scratch_shapes=[pltpu.VMEM((2, 128, _ENC), jnp.float32)],
    )(h1s, agg2s, w_self2, b2_row, batchf, wh1a, wh1b, bh1_row, wh2p, bh2b)


def _sc_body(t1_hbm, gidx1_hbm, dst1_hbm, t2_hbm, gidx2_hbm, dst2_hbm,
             zeros_hbm, out1_hbm, out2_hbm, gidx_v, dst_v, rows0_v, rows1_v,
             acc_sh, semg0, semg1, sem):
    cid = lax.axis_index("c")
    sid = lax.axis_index("s")

    pltpu.sync_copy(zeros_hbm, acc_sh.at[pl.ds(sid * _ZROWS, _ZROWS)])
    plsc.subcore_barrier()

    def do_edges(t_hbm, gidx_hbm, dst_hbm):
        rows = (rows0_v, rows1_v)
        sg = (semg0, semg1)

        def group(g, carry):
            pltpu.sync_copy(gidx_hbm.at[sid, g], gidx_v)
            pltpu.sync_copy(dst_hbm.at[sid, g], dst_v)
            gath = {}
            for j in range(_GRP):
                b = j % 2
                gath[j] = pltpu.async_copy(t_hbm.at[gidx_v.at[j]], rows[b],
                                           sg[b])
                if j >= 1:
                    gath[j - 1].wait()
                    pltpu.sync_copy(rows[1 - b], acc_sh.at[dst_v.at[j - 1]],
                                    add=True)
            last = _GRP - 1
            gath[last].wait()
            pltpu.sync_copy(rows[last % 2], acc_sh.at[dst_v.at[last]],
                            add=True)
            return carry

        lax.fori_loop(0, _NGRP, group, 0)

    @pl.when(cid == 0)
    def _():
        do_edges(t1_hbm, gidx1_hbm, dst1_hbm)

    @pl.when(cid == 1)
    def _():
        do_edges(t2_hbm, gidx2_hbm, dst2_hbm)

    plsc.subcore_barrier()

    @pl.when(cid == 0)
    def _():
        pltpu.sync_copy(acc_sh.at[pl.ds(sid * _ZROWS, _ZROWS)],
                        out1_hbm.at[pl.ds(sid * _ZROWS, _ZROWS)])

    @pl.when(cid == 1)
    def _():
        pltpu.sync_copy(acc_sh.at[pl.ds(sid * _ZROWS, _ZROWS)],
                        out2_hbm.at[pl.ds(sid * _ZROWS, _ZROWS)])


def _sc_agg(t1, t2, gidx1, dst1, gidx2, dst2, zeros):
    mesh = plsc.VectorSubcoreMesh(core_axis_name="c", subcore_axis_name="s")
    f = pl.kernel(
        _sc_body,
        out_type=(jax.ShapeDtypeStruct((_NACC, _H), jnp.float32),
                  jax.ShapeDtypeStruct((_NACC, _H), jnp.float32)),
        mesh=mesh,
        scratch_types=[
            pltpu.VMEM((_GRP, _CH), jnp.int32),
            pltpu.VMEM((_GRP, _CH), jnp.int32),
            pltpu.VMEM((_CH, _H), jnp.float32),
            pltpu.VMEM((_CH, _H), jnp.float32),
            pltpu.VMEM_SHARED((_NACC, _H), jnp.float32),
            pltpu.SemaphoreType.DMA,
            pltpu.SemaphoreType.DMA,
            pltpu.SemaphoreType.DMA,
        ],
    )
    o1, o2 = f(t1, gidx1, dst1, t2, gidx2, dst2, zeros)
    return o1[:_N], o2[:_N]


def _prep_edges(edge_index, edge_type):
    src = edge_index[0].astype(jnp.int32)
    dst = edge_index[1].astype(jnp.int32)
    gidx = src * _R + edge_type.astype(jnp.int32)
    order = jnp.argsort(gidx)
    gidx = gidx[order]
    dst = dst[order]
    pad = _EPAD - _E
    gidxp = jnp.concatenate([gidx, jnp.zeros((pad,), jnp.int32)])
    dstp = jnp.concatenate([dst, jnp.full((pad,), _NACC - 1, jnp.int32)])
    return (gidxp.reshape(_NSUB, _NGRP, _GRP, _CH),
            dstp.reshape(_NSUB, _NGRP, _GRP, _CH))


def kernel(x1, edge_index1, edge_type1, batch1, x2, edge_index2, edge_type2,
           batch2, W_rel1, W_self1, b1, W_rel2, W_self2, b2, Wh1, bh1, Wh2,
           bh2):
    f32 = jnp.float32
    xs = jnp.concatenate([x1, x2], axis=0)
    wflat1 = jnp.transpose(W_rel1, (1, 0, 2)).reshape(_D, _R * _H)
    wflat2 = jnp.transpose(W_rel2, (1, 0, 2)).reshape(_H, _R * _H)

    g1, d1 = _prep_edges(edge_index1, edge_type1)
    g2, d2 = _prep_edges(edge_index2, edge_type2)
    zeros = jnp.zeros((_ZROWS, _H), f32)

    t = _row_matmul(xs, wflat1)
    agg1_1, agg1_2 = _sc_agg(t[:_N].reshape(_N * _R, _H),
                             t[_N:].reshape(_N * _R, _H),
                             g1, d1, g2, d2, zeros)
    aggs1 = jnp.concatenate([agg1_1, agg1_2], axis=0)
    h1s, u = _combine_stage(xs, aggs1, W_self1, b1.reshape(1, _H), wflat2)
    agg2_1, agg2_2 = _sc_agg(u[:_N].reshape(_N * _R, _H),
                             u[_N:].reshape(_N * _R, _H),
                             g1, d1, g2, d2, zeros)
    aggs2 = jnp.concatenate([agg2_1, agg2_2], axis=0)

    batchf = jnp.broadcast_to(
        jnp.concatenate([batch1, batch2]).astype(f32)[:, None], (2 * _N, 128))
    wh1a = Wh1[:_ENC]
    wh1b = Wh1[_ENC:]
    wh2p = jnp.pad(Wh2, ((0, 0), (0, 127)))
    bh2b = jnp.broadcast_to(bh2.reshape(1, 1), (1, 128))
    outm = _final_stage(h1s, aggs2, W_self2, b2.reshape(1, _H), batchf,
                        wh1a, wh1b, bh1.reshape(1, 128), wh2p, bh2b)
    return outm[:_B, 0]


# speedup vs baseline: 1.0746x; 1.0746x over previous
import jax
import jax.numpy as jnp
from jax import lax
from jax.experimental import pallas as pl
from jax.experimental.pallas import tpu as pltpu
from jax.experimental.pallas import tpu_sc as plsc

_N = 10000
_E = 320000
_D = 128
_R = 7
_H = 128
_B = 16
_ENC = 2 * _H

_NSUB = 16
_WRK = 32
_CH = 128
_GRPS = (56, 24)
_GRPMAX = 56
_NCH = sum(_GRPS)
_EPAD = _WRK * _NCH * _CH
_ZROWS = 640
_NACC = _NSUB * _ZROWS

_RB = 1000


def _matmul_body(x_ref, w_ref, o_ref):
    o_ref[...] = jnp.dot(x_ref[...], w_ref[...], preferred_element_type=jnp.float32)


def _row_matmul(x, w):
    m, k = x.shape
    n = w.shape[1]
    return pl.pallas_call(
        _matmul_body,
        grid=(m // _RB,),
        in_specs=[
            pl.BlockSpec((_RB, k), lambda b: (b, 0)),
            pl.BlockSpec((k, n), lambda b: (0, 0)),
        ],
        out_specs=pl.BlockSpec((_RB, n), lambda b: (b, 0)),
        out_shape=jax.ShapeDtypeStruct((m, n), jnp.float32),
    )(x, w)


def _combine_body(x_ref, aga_ref, agb_ref, ws_ref, b_ref, wf_ref, h_ref, t_ref):
    h = jnp.maximum(
        jnp.dot(x_ref[...], ws_ref[...], preferred_element_type=jnp.float32)
        + b_ref[...] + aga_ref[...] + agb_ref[...], 0.0)
    h_ref[...] = h
    t_ref[...] = jnp.dot(h, wf_ref[...], preferred_element_type=jnp.float32)


def _combine_stage(x, agg_pair, w_self, b_row, w_flat):
    m, k = x.shape
    n = w_flat.shape[1]
    return pl.pallas_call(
        _combine_body,
        grid=(m // _RB,),
        in_specs=[
            pl.BlockSpec((_RB, k), lambda b: (b, 0)),
            pl.BlockSpec((_RB, _H), lambda b: (b, 0)),
            pl.BlockSpec((_RB, _H), lambda b: (b, 0)),
            pl.BlockSpec((k, _H), lambda b: (0, 0)),
            pl.BlockSpec((1, _H), lambda b: (0, 0)),
            pl.BlockSpec((_H, n), lambda b: (0, 0)),
        ],
        out_specs=[
            pl.BlockSpec((_RB, _H), lambda b: (b, 0)),
            pl.BlockSpec((_RB, n), lambda b: (b, 0)),
        ],
        out_shape=[
            jax.ShapeDtypeStruct((m, _H), jnp.float32),
            jax.ShapeDtypeStruct((m, n), jnp.float32),
        ],
    )(x, agg_pair[0], agg_pair[1], w_self, b_row, w_flat)


def _final_body(h1_ref, agg_ref, ws_ref, b_ref, bf_ref, wh1a_ref, wh1b_ref,
                bh1_ref, wh2_ref, bh2_ref, o_ref, gacc):
    b = pl.program_id(0)
    nb = pl.num_programs(0)
    g = b // (nb // 2)

    @pl.when(b == 0)
    def _():
        gacc[0] = jnp.zeros_like(gacc[0])
        gacc[1] = jnp.zeros_like(gacc[1])

    h1 = h1_ref[...]
    h2 = jnp.maximum(
        jnp.dot(h1, ws_ref[...], preferred_element_type=jnp.float32)
        + b_ref[...] + agg_ref[...], 0.0)
    feats = jnp.concatenate([h1, h2], axis=1)
    lane = lax.broadcasted_iota(jnp.int32, (_RB, 128), 1).astype(jnp.float32)
    oh = (bf_ref[...] == lane).astype(jnp.float32)
    contrib = lax.dot_general(oh, feats, (((0,), (0,)), ((), ())),
                              preferred_element_type=jnp.float32)
    gacc[g] = gacc[g] + contrib

    @pl.when(b == nb - 1)
    def _():
        hid = jnp.maximum(
            jnp.dot(gacc[0], wh1a_ref[...], preferred_element_type=jnp.float32)
            + jnp.dot(gacc[1], wh1b_ref[...], preferred_element_type=jnp.float32)
            + bh1_ref[...], 0.0)
        o_ref[...] = jnp.dot(hid, wh2_ref[...],
                             preferred_element_type=jnp.float32) + bh2_ref[...]


def _final_stage(h1s, agg2s, w_self2, b2_row, batchf, wh1a, wh1b, bh1_row,
                 wh2p, bh2b):
    m = h1s.shape[0]
    return pl.pallas_call(
        _final_body,
        grid=(m // _RB,),
        in_specs=[
            pl.BlockSpec((_RB, _H), lambda b: (b, 0)),
            pl.BlockSpec((_RB, _H), lambda b: (b, 0)),
            pl.BlockSpec((_H, _H), lambda b: (0, 0)),
            pl.BlockSpec((1, _H), lambda b: (0, 0)),
            pl.BlockSpec((_RB, 128), lambda b: (b, 0)),
            pl.BlockSpec((_ENC, 128), lambda b: (0, 0)),
            pl.BlockSpec((_ENC, 128), lambda b: (0, 0)),
            pl.BlockSpec((1, 128), lambda b: (0, 0)),
            pl.BlockSpec((128, 128), lambda b: (0, 0)),
            pl.BlockSpec((1, 128), lambda b: (0, 0)),
        ],
        out_specs=pl.BlockSpec((128, 128), lambda b: (0, 0)),
        out_shape=jax.ShapeDtypeStruct((128, 128), jnp.float32),
        scratch_shapes=[pltpu.VMEM((2, 128, _ENC), jnp.float32)],
    )(h1s, agg2s, w_self2, b2_row, batchf, wh1a, wh1b, bh1_row, wh2p, bh2b)


def _sc_body(t_hbm, gidx_hbm, dst_hbm, zeros_hbm, out_hbm, gidx_v, dst_v,
             rows0_v, rows1_v, acc_sh, semg0, semg1, semst):
    cid = lax.axis_index("c")
    sid = lax.axis_index("s")
    wid = cid * _NSUB + sid

    pltpu.sync_copy(zeros_hbm, acc_sh.at[pl.ds(sid * _ZROWS, _ZROWS)])
    plsc.subcore_barrier()

    rows = (rows0_v, rows1_v)
    sg = (semg0, semg1)

    off = 0
    for grp in _GRPS:
        st1 = pltpu.async_copy(gidx_hbm.at[wid, pl.ds(off, grp)],
                               gidx_v.at[pl.ds(0, grp)], semst)
        st2 = pltpu.async_copy(dst_hbm.at[wid, pl.ds(off, grp)],
                               dst_v.at[pl.ds(0, grp)], semst)
        st1.wait()
        st2.wait()
        gath = {}
        for j in range(grp):
            b = j % 2
            gath[j] = pltpu.async_copy(t_hbm.at[gidx_v.at[j]], rows[b], sg[b])
            if j >= 1:
                gath[j - 1].wait()
                pltpu.sync_copy(rows[1 - b], acc_sh.at[dst_v.at[j - 1]],
                                add=True)
        last = grp - 1
        gath[last].wait()
        pltpu.sync_copy(rows[last % 2], acc_sh.at[dst_v.at[last]], add=True)
        off += grp

    plsc.subcore_barrier()
    pltpu.sync_copy(acc_sh.at[pl.ds(sid * _ZROWS, _ZROWS)],
                    out_hbm.at[cid, pl.ds(sid * _ZROWS, _ZROWS)])


def _sc_agg(t, gidx, dst, zeros):
    mesh = plsc.VectorSubcoreMesh(core_axis_name="c", subcore_axis_name="s")
    f = pl.kernel(
        _sc_body,
        out_type=jax.ShapeDtypeStruct((2, _NACC, _H), jnp.float32),
        mesh=mesh,
        scratch_types=[
            pltpu.VMEM((_GRPMAX, _CH), jnp.int32),
            pltpu.VMEM((_GRPMAX, _CH), jnp.int32),
            pltpu.VMEM((_CH, _H), jnp.float32),
            pltpu.VMEM((_CH, _H), jnp.float32),
            pltpu.VMEM_SHARED((_NACC, _H), jnp.float32),
            pltpu.SemaphoreType.DMA,
            pltpu.SemaphoreType.DMA,
            pltpu.SemaphoreType.DMA,
        ],
    )
    return f(t, gidx, dst, zeros)


def _prep_edges(edge_index, edge_type):
    src = edge_index[0].astype(jnp.int32)
    dst = edge_index[1].astype(jnp.int32)
    gidx = src * _R + edge_type.astype(jnp.int32)
    pad = _EPAD - _E
    gidxp = jnp.concatenate([gidx, jnp.zeros((pad,), jnp.int32)])
    dstp = jnp.concatenate([dst, jnp.full((pad,), _NACC - 1, jnp.int32)])
    return (gidxp.reshape(_WRK, _NCH, _CH), dstp.reshape(_WRK, _NCH, _CH))


def kernel(x1, edge_index1, edge_type1, batch1, x2, edge_index2, edge_type2,
           batch2, W_rel1, W_self1, b1, W_rel2, W_self2, b2, Wh1, bh1, Wh2,
           bh2):
    f32 = jnp.float32
    wflat1 = jnp.transpose(W_rel1, (1, 0, 2)).reshape(_D, _R * _H)
    wflat2 = jnp.transpose(W_rel2, (1, 0, 2)).reshape(_H, _R * _H)

    g1, d1 = _prep_edges(edge_index1, edge_type1)
    g2, d2 = _prep_edges(edge_index2, edge_type2)
    zeros = jnp.zeros((_ZROWS, _H), f32)
    b1r = b1.reshape(1, _H)

    ta1 = _row_matmul(x1, wflat1)
    ta2 = _row_matmul(x2, wflat1)
    p1 = _sc_agg(ta1.reshape(_N * _R, _H), g1, d1, zeros)
    p2 = _sc_agg(ta2.reshape(_N * _R, _H), g2, d2, zeros)
    h11, u1 = _combine_stage(x1, p1, W_self1, b1r, wflat2)
    q1 = _sc_agg(u1.reshape(_N * _R, _H), g1, d1, zeros)
    h12, u2 = _combine_stage(x2, p2, W_self1, b1r, wflat2)
    q2 = _sc_agg(u2.reshape(_N * _R, _H), g2, d2, zeros)

    agg21 = q1[0, :_N] + q1[1, :_N]
    agg22 = q2[0, :_N] + q2[1, :_N]
    h1s = jnp.concatenate([h11, h12], axis=0)
    aggs2 = jnp.concatenate([agg21, agg22], axis=0)
    batchf = jnp.broadcast_to(
        jnp.concatenate([batch1, batch2]).astype(f32)[:, None], (2 * _N, 128))
    wh1a = Wh1[:_ENC]
    wh1b = Wh1[_ENC:]
    wh2p = jnp.pad(Wh2, ((0, 0), (0, 127)))
    bh2b = jnp.broadcast_to(bh2.reshape(1, 1), (1, 128))
    outm = _final_stage(h1s, aggs2, W_self2, b2.reshape(1, _H), batchf,
                        wh1a, wh1b, bh1.reshape(1, 128), wh2p, bh2b)
    return outm[:_B, 0]


# speedup vs baseline: 1.5246x; 1.4188x over previous
import functools

import jax
import jax.numpy as jnp
from jax import lax
from jax.experimental import pallas as pl
from jax.experimental.pallas import tpu as pltpu
from jax.experimental.pallas import tpu_sc as plsc

_N = 10000
_E = 320000
_D = 128
_R = 7
_H = 128
_B = 16
_ENC = 2 * _H

_NSUB = 16
_CH = 128
_GRPS = (56, 56, 48)
_GRPMAX = 56
_NCH = sum(_GRPS)
_EPAD = _NSUB * _NCH * _CH
_ZROWS = 640
_NACC = _NSUB * _ZROWS
_ROWS_PER_TILE = _N // _NSUB

_RB = 1000


def _matmul_body(x_ref, w_ref, o_ref):
    o_ref[...] = jnp.dot(x_ref[...], w_ref[...], preferred_element_type=jnp.float32)


def _row_matmul(x, w):
    m, k = x.shape
    n = w.shape[1]
    return pl.pallas_call(
        _matmul_body,
        grid=(m // _RB,),
        in_specs=[
            pl.BlockSpec((_RB, k), lambda b: (b, 0)),
            pl.BlockSpec((k, n), lambda b: (0, 0)),
        ],
        out_specs=pl.BlockSpec((_RB, n), lambda b: (b, 0)),
        out_shape=jax.ShapeDtypeStruct((m, n), jnp.float32),
    )(x, w)


def _combine_body(x_ref, agg_ref, ws_ref, b_ref, wf_ref, h_ref, t_ref):
    h = jnp.maximum(
        jnp.dot(x_ref[...], ws_ref[...], preferred_element_type=jnp.float32)
        + b_ref[...] + agg_ref[...], 0.0)
    h_ref[...] = h
    t_ref[...] = jnp.dot(h, wf_ref[...], preferred_element_type=jnp.float32)


def _combine_stage(x, agg, w_self, b_row, w_flat):
    m, k = x.shape
    n = w_flat.shape[1]
    return pl.pallas_call(
        _combine_body,
        grid=(m // _RB,),
        in_specs=[
            pl.BlockSpec((_RB, k), lambda b: (b, 0)),
            pl.BlockSpec((_RB, _H), lambda b: (b, 0)),
            pl.BlockSpec((k, _H), lambda b: (0, 0)),
            pl.BlockSpec((1, _H), lambda b: (0, 0)),
            pl.BlockSpec((_H, n), lambda b: (0, 0)),
        ],
        out_specs=[
            pl.BlockSpec((_RB, _H), lambda b: (b, 0)),
            pl.BlockSpec((_RB, n), lambda b: (b, 0)),
        ],
        out_shape=[
            jax.ShapeDtypeStruct((m, _H), jnp.float32),
            jax.ShapeDtypeStruct((m, n), jnp.float32),
        ],
    )(x, agg, w_self, b_row, w_flat)


def _final_body(h1_ref, agg_ref, ws_ref, b_ref, bf_ref, wh1a_ref, wh1b_ref,
                bh1_ref, wh2_ref, bh2_ref, o_ref, gacc):
    b = pl.program_id(0)
    nb = pl.num_programs(0)
    g = b // (nb // 2)

    @pl.when(b == 0)
    def _():
        gacc[0] = jnp.zeros_like(gacc[0])
        gacc[1] = jnp.zeros_like(gacc[1])

    h1 = h1_ref[...]
    h2 = jnp.maximum(
        jnp.dot(h1, ws_ref[...], preferred_element_type=jnp.float32)
        + b_ref[...] + agg_ref[...], 0.0)
    feats = jnp.concatenate([h1, h2], axis=1)
    lane = lax.broadcasted_iota(jnp.int32, (_RB, 128), 1).astype(jnp.float32)
    oh = (bf_ref[...] == lane).astype(jnp.float32)
    contrib = lax.dot_general(oh, feats, (((0,), (0,)), ((), ())),
                              preferred_element_type=jnp.float32,
                              precision=lax.Precision.HIGHEST)
    gacc[g] = gacc[g] + contrib

    @pl.when(b == nb - 1)
    def _():
        hid = jnp.maximum(
            jnp.dot(gacc[0], wh1a_ref[...], preferred_element_type=jnp.float32)
            + jnp.dot(gacc[1], wh1b_ref[...], preferred_element_type=jnp.float32)
            + bh1_ref[...], 0.0)
        o_ref[...] = jnp.dot(hid, wh2_ref[...],
                             preferred_element_type=jnp.float32) + bh2_ref[...]


def _final_stage(h1s, agg2s, w_self2, b2_row, batchf, wh1a, wh1b, bh1_row,
                 wh2p, bh2b):
    m = h1s.shape[0]
    return pl.pallas_call(
        _final_body,
        grid=(m // _RB,),
        in_specs=[
            pl.BlockSpec((_RB, _H), lambda b: (b, 0)),
            pl.BlockSpec((_RB, _H), lambda b: (b, 0)),
            pl.BlockSpec((_H, _H), lambda b: (0, 0)),
            pl.BlockSpec((1, _H), lambda b: (0, 0)),
            pl.BlockSpec((_RB, 128), lambda b: (b, 0)),
            pl.BlockSpec((_ENC, 128), lambda b: (0, 0)),
            pl.BlockSpec((_ENC, 128), lambda b: (0, 0)),
            pl.BlockSpec((1, 128), lambda b: (0, 0)),
            pl.BlockSpec((128, 128), lambda b: (0, 0)),
            pl.BlockSpec((1, 128), lambda b: (0, 0)),
        ],
        out_specs=pl.BlockSpec((128, 128), lambda b: (0, 0)),
        out_shape=jax.ShapeDtypeStruct((128, 128), jnp.float32),
        scratch_shapes=[pltpu.VMEM((2, 128, _ENC), jnp.float32)],
    )(h1s, agg2s, w_self2, b2_row, batchf, wh1a, wh1b, bh1_row, wh2p, bh2b)


def _sc_body(t1_hbm, gidx1_hbm, dst1_hbm, t2_hbm, gidx2_hbm, dst2_hbm,
             zeros_hbm, out1_hbm, out2_hbm, gidx_v, dst_v, rows0_v, rows1_v,
             acc_sh, semg0, semg1, semst):
    cid = lax.axis_index("c")
    sid = lax.axis_index("s")

    pltpu.sync_copy(zeros_hbm, acc_sh.at[pl.ds(sid * _ZROWS, _ZROWS)])
    plsc.subcore_barrier()

    def do_edges(t_hbm, gidx_hbm, dst_hbm):
        rows = (rows0_v, rows1_v)
        sg = (semg0, semg1)

        off = 0
        for grp in _GRPS:
            st1 = pltpu.async_copy(gidx_hbm.at[sid, pl.ds(off, grp)],
                                   gidx_v.at[pl.ds(0, grp)], semst)
            st2 = pltpu.async_copy(dst_hbm.at[sid, pl.ds(off, grp)],
                                   dst_v.at[pl.ds(0, grp)], semst)
            st1.wait()
            st2.wait()
            gath = {}
            for j in range(grp):
                b = j % 2
                gath[j] = pltpu.async_copy(t_hbm.at[gidx_v.at[j]], rows[b],
                                           sg[b])
                if j >= 1:
                    gath[j - 1].wait()
                    pltpu.sync_copy(rows[1 - b], acc_sh.at[dst_v.at[j - 1]],
                                    add=True)
            last = grp - 1
            gath[last].wait()
            pltpu.sync_copy(rows[last % 2], acc_sh.at[dst_v.at[last]],
                            add=True)
            off += grp

    @pl.when(cid == 0)
    def _():
        do_edges(t1_hbm, gidx1_hbm, dst1_hbm)

    @pl.when(cid == 1)
    def _():
        do_edges(t2_hbm, gidx2_hbm, dst2_hbm)

    plsc.subcore_barrier()

    @pl.when(cid == 0)
    def _():
        pltpu.sync_copy(acc_sh.at[pl.ds(sid * _ZROWS, _ZROWS)],
                        out1_hbm.at[pl.ds(sid * _ZROWS, _ZROWS)])

    @pl.when(cid == 1)
    def _():
        pltpu.sync_copy(acc_sh.at[pl.ds(sid * _ZROWS, _ZROWS)],
                        out2_hbm.at[pl.ds(sid * _ZROWS, _ZROWS)])


def _sc_agg(t1, t2, gidx1, dst1, gidx2, dst2, zeros):
    mesh = plsc.VectorSubcoreMesh(core_axis_name="c", subcore_axis_name="s")
    f = pl.kernel(
        _sc_body,
        out_type=(jax.ShapeDtypeStruct((_NACC, _H), jnp.float32),
                  jax.ShapeDtypeStruct((_NACC, _H), jnp.float32)),
        mesh=mesh,
        scratch_types=[
            pltpu.VMEM((_GRPMAX, _CH), jnp.int32),
            pltpu.VMEM((_GRPMAX, _CH), jnp.int32),
            pltpu.VMEM((_CH, _H), jnp.float32),
            pltpu.VMEM((_CH, _H), jnp.float32),
            pltpu.VMEM_SHARED((_NACC, _H), jnp.float32),
            pltpu.SemaphoreType.DMA,
            pltpu.SemaphoreType.DMA,
            pltpu.SemaphoreType.DMA,
        ],
    )
    o1, o2 = f(t1, gidx1, dst1, t2, gidx2, dst2, zeros)
    return o1[:_N], o2[:_N]


def _prep_edges(edge_index, edge_type):
    src = edge_index[0].astype(jnp.int32)
    dst = edge_index[1].astype(jnp.int32)
    gidx = src * _R + edge_type.astype(jnp.int32)
    pad = _EPAD - _E
    gidxp = jnp.concatenate([gidx, jnp.zeros((pad,), jnp.int32)])
    dstp = jnp.concatenate([dst, jnp.full((pad,), _NACC - 1, jnp.int32)])
    return (gidxp.reshape(_NSUB, _NCH, _CH),
            dstp.reshape(_NSUB, _NCH, _CH))


def kernel(x1, edge_index1, edge_type1, batch1, x2, edge_index2, edge_type2,
           batch2, W_rel1, W_self1, b1, W_rel2, W_self2, b2, Wh1, bh1, Wh2,
           bh2):
    f32 = jnp.float32
    xs = jnp.concatenate([x1, x2], axis=0)
    wflat1 = jnp.transpose(W_rel1, (1, 0, 2)).reshape(_D, _R * _H)
    wflat2 = jnp.transpose(W_rel2, (1, 0, 2)).reshape(_H, _R * _H)

    g1, d1 = _prep_edges(edge_index1, edge_type1)
    g2, d2 = _prep_edges(edge_index2, edge_type2)
    zeros = jnp.zeros((_ZROWS, _H), f32)

    t = _row_matmul(xs, wflat1)
    agg1_1, agg1_2 = _sc_agg(t[:_N].reshape(_N * _R, _H),
                             t[_N:].reshape(_N * _R, _H),
                             g1, d1, g2, d2, zeros)
    aggs1 = jnp.concatenate([agg1_1, agg1_2], axis=0)
    h1s, u = _combine_stage(xs, aggs1, W_self1, b1.reshape(1, _H), wflat2)
    agg2_1, agg2_2 = _sc_agg(u[:_N].reshape(_N * _R, _H),
                             u[_N:].reshape(_N * _R, _H),
                             g1, d1, g2, d2, zeros)
    aggs2 = jnp.concatenate([agg2_1, agg2_2], axis=0)

    batchf = jnp.broadcast_to(
        jnp.concatenate([batch1, batch2]).astype(f32)[:, None], (2 * _N, 128))
    wh1a = Wh1[:_ENC]
    wh1b = Wh1[_ENC:]
    wh2p = jnp.pad(Wh2, ((0, 0), (0, 127)))
    bh2b = jnp.broadcast_to(bh2.reshape(1, 1), (1, 128))
    outm = _final_stage(h1s, aggs2, W_self2, b2.reshape(1, _H), batchf,
                        wh1a, wh1b, bh1.reshape(1, 128), wh2p, bh2b)
    return outm[:_B, 0]
